# baseline (speedup 1.0000x reference)
import functools

import jax
import jax.numpy as jnp
from jax import lax
from jax.experimental import pallas as pl
from jax.experimental.pallas import tpu as pltpu
from jax.experimental.pallas import tpu_sc as plsc

D = 32
DW = 16
A = 64
HIST = 50
PAD = 56
GROUP = 2
NC, NS = 2, 16
NW = NC * NS


def _sc_pool(obj_pad_flat, table_pk, B):
    E = B // NW
    NG = E // GROUP
    GI = GROUP * PAD
    NBUF = 8

    mesh = plsc.VectorSubcoreMesh(
        core_axis_name="c", subcore_axis_name="s",
        num_cores=NC, num_subcores=NS)

    @functools.partial(
        pl.kernel,
        out_type=jax.ShapeDtypeStruct((B * D,), jnp.float32),
        mesh=mesh,
        scratch_types=[
            pltpu.VMEM((E * PAD,), jnp.int32),
            pltpu.VMEM((NBUF, GI, DW), jnp.int32),
            pltpu.VMEM((E * D,), jnp.float32),
            [pltpu.SemaphoreType.DMA] * NBUF,
        ],
        compiler_params=pltpu.CompilerParams(use_tc_tiling_on_sc=False,
                                             needs_layout_passes=False),
    )
    def sc_kernel(obj_hbm, table_hbm, out_hbm, idx_v, rows_v, out_v, sems):
        wid = lax.axis_index("c") * NS + lax.axis_index("s")

        pltpu.sync_copy(obj_hbm.at[pl.ds(wid * (E * PAD), E * PAD)], idx_v)

        def gather(gg, buf):
            off = pl.multiple_of(gg * GI, 8)
            return pltpu.make_async_copy(
                table_hbm.at[idx_v.at[pl.ds(off, GI)]],
                rows_v.at[buf], sems[buf])

        himask = jnp.full((16,), -65536, dtype=jnp.int32)

        def unpack_row(w):
            lo = plsc.bitcast(jnp.left_shift(w, 16), jnp.float32)
            hi = plsc.bitcast(jnp.bitwise_and(w, himask), jnp.float32)
            return lo, hi

        def reduce_group(gg, buf):
            for e in range(GROUP):
                r0 = e * PAD
                acc_lo, acc_hi = unpack_row(rows_v[buf, r0, :])
                for j in range(1, HIST):
                    lo, hi = unpack_row(rows_v[buf, r0 + j, :])
                    acc_lo = acc_lo + lo
                    acc_hi = acc_hi + hi
                row = (gg * GROUP + e) * D
                out_v[pl.ds(pl.multiple_of(row, D), 16)] = acc_lo
                out_v[pl.ds(pl.multiple_of(row + 16, 16), 16)] = acc_hi

        for b in range(NBUF):
            gather(b, b).start()

        @pl.loop(0, NG, step=NBUF)
        def _outer(g):
            for buf in range(NBUF):
                gg = g + buf
                gather(gg, buf).wait()

                @pl.when(gg + NBUF < NG)
                def _start_next():
                    gather(gg + NBUF, buf).start()

                reduce_group(gg, buf)

        pltpu.sync_copy(out_v, out_hbm.at[pl.ds(wid * (E * D), E * D)])

    return sc_kernel(obj_pad_flat, table_pk)


def _tc_project(vec, v, g2, b2):
    B = vec.shape[0]

    def body(vec_ref, v_ref, g_ref, b_ref, o_ref):
        vv = v_ref[...]
        norm = jnp.sqrt(jnp.sum(vv * vv, axis=1, keepdims=True))
        w = (g_ref[...] / norm) * (1.0 / HIST) * vv
        o_ref[...] = lax.dot_general(
            vec_ref[...], w, (((1,), (1,)), ((), ())),
            preferred_element_type=jnp.float32) + b_ref[...]

    return pl.pallas_call(
        body,
        out_shape=jax.ShapeDtypeStruct((B, A), jnp.float32),
    )(vec, v, g2, b2)


_PKC = 2048


def _pack_table(table):
    V = table.shape[0]
    tbt = table.T

    def body(x_ref, o_ref):
        x = x_ref[...]
        ul = lax.bitcast_convert_type(x[:DW, :], jnp.uint32)
        uh = lax.bitcast_convert_type(x[DW:, :], jnp.uint32)
        rl = (ul + jnp.uint32(0x7FFF) + ((ul >> 16) & jnp.uint32(1))) >> 16
        th = uh + jnp.uint32(0x7FFF) + ((uh >> 16) & jnp.uint32(1))
        w = (th & jnp.uint32(0xFFFF0000)) | rl
        o_ref[...] = lax.bitcast_convert_type(w, jnp.int32)

    pk = pl.pallas_call(
        body,
        grid=(pl.cdiv(V, _PKC),),
        in_specs=[pl.BlockSpec((D, _PKC), lambda i: (0, i))],
        out_specs=pl.BlockSpec((DW, _PKC), lambda i: (0, i)),
        out_shape=jax.ShapeDtypeStruct((DW, V), jnp.int32),
    )(tbt)
    return pk.T


def kernel(obj, table, v, g, b):
    B, H = obj.shape
    obj_p = jnp.pad(obj.astype(jnp.int32), ((0, 0), (0, PAD - H)))
    vec = _sc_pool(obj_p.reshape(-1), _pack_table(table), B)
    vec = vec.reshape(B, D)
    return _tc_project(vec, v, g.reshape(A, 1), b.reshape(1, A))


# speedup vs baseline: 1.4320x; 1.4320x over previous
import functools

import jax
import jax.numpy as jnp
from jax import lax
from jax.experimental import pallas as pl
from jax.experimental.pallas import tpu as pltpu
from jax.experimental.pallas import tpu_sc as plsc

D = 32
DW = 16
A = 64
HIST = 50
PAD = 56
GROUP = 2
NC, NS = 2, 16
NW = NC * NS


def _sc_pool(obj_pad_flat, table_pk, B):
    E = B // NW
    NG = E // GROUP
    GI = GROUP * PAD
    NBUF = 8

    mesh = plsc.VectorSubcoreMesh(
        core_axis_name="c", subcore_axis_name="s",
        num_cores=NC, num_subcores=NS)

    @functools.partial(
        pl.kernel,
        out_type=jax.ShapeDtypeStruct((B * D,), jnp.float32),
        mesh=mesh,
        scratch_types=[
            pltpu.VMEM((E * PAD,), jnp.int32),
            pltpu.VMEM((NBUF, GI, DW), jnp.int32),
            pltpu.VMEM((E * D,), jnp.float32),
            [pltpu.SemaphoreType.DMA] * NBUF,
        ],
        compiler_params=pltpu.CompilerParams(use_tc_tiling_on_sc=False,
                                             needs_layout_passes=False),
    )
    def sc_kernel(obj_hbm, table_hbm, out_hbm, idx_v, rows_v, out_v, sems):
        wid = lax.axis_index("c") * NS + lax.axis_index("s")

        pltpu.sync_copy(obj_hbm.at[pl.ds(wid * (E * PAD), E * PAD)], idx_v)

        def gather(gg, buf):
            off = pl.multiple_of(gg * GI, 8)
            return pltpu.make_async_copy(
                table_hbm.at[idx_v.at[pl.ds(off, GI)]],
                rows_v.at[buf], sems[buf])

        himask = jnp.full((16,), -65536, dtype=jnp.int32)

        def unpack_row(w):
            lo = plsc.bitcast(jnp.left_shift(w, 16), jnp.float32)
            hi = plsc.bitcast(jnp.bitwise_and(w, himask), jnp.float32)
            return lo, hi

        def reduce_group(gg, buf):
            for e in range(GROUP):
                r0 = e * PAD
                acc_lo, acc_hi = unpack_row(rows_v[buf, r0, :])
                for j in range(1, HIST):
                    lo, hi = unpack_row(rows_v[buf, r0 + j, :])
                    acc_lo = acc_lo + lo
                    acc_hi = acc_hi + hi
                row = (gg * GROUP + e) * D
                out_v[pl.ds(pl.multiple_of(row, D), 16)] = acc_lo
                out_v[pl.ds(pl.multiple_of(row + 16, 16), 16)] = acc_hi

        for b in range(NBUF):
            gather(b, b).start()

        @pl.loop(0, NG, step=NBUF)
        def _outer(g):
            for buf in range(NBUF):
                gg = g + buf
                gather(gg, buf).wait()

                @pl.when(gg + NBUF < NG)
                def _start_next():
                    gather(gg + NBUF, buf).start()

                reduce_group(gg, buf)

        pltpu.sync_copy(out_v, out_hbm.at[pl.ds(wid * (E * D), E * D)])

    return sc_kernel(obj_pad_flat, table_pk)


def _tc_project(vec, v, g2, b2):
    B = vec.shape[0]

    def body(vec_ref, v_ref, g_ref, b_ref, o_ref):
        vv = v_ref[...]
        norm = jnp.sqrt(jnp.sum(vv * vv, axis=1, keepdims=True))
        w = (g_ref[...] / norm) * (1.0 / HIST) * vv
        o_ref[...] = lax.dot_general(
            vec_ref[...], w, (((1,), (1,)), ((), ())),
            preferred_element_type=jnp.float32) + b_ref[...]

    return pl.pallas_call(
        body,
        out_shape=jax.ShapeDtypeStruct((B, A), jnp.float32),
    )(vec, v, g2, b2)


_PKC = 2048


def _pack_table(table):
    V = table.shape[0]
    tbt = table.T

    def body(x_ref, o_ref):
        x = x_ref[...]
        ul = lax.bitcast_convert_type(x[:DW, :], jnp.uint32)
        uh = lax.bitcast_convert_type(x[DW:, :], jnp.uint32)
        rl = (ul + jnp.uint32(0x7FFF) + ((ul >> 16) & jnp.uint32(1))) >> 16
        th = uh + jnp.uint32(0x7FFF) + ((uh >> 16) & jnp.uint32(1))
        w = (th & jnp.uint32(0xFFFF0000)) | rl
        wt = jnp.transpose(lax.bitcast_convert_type(w, jnp.int32))
        w3 = wt.reshape(_PKC // 8, 8, DW)
        for j in range(8):
            o_ref[:, pl.ds(j * DW, DW)] = w3[:, j, :]

    rows_per = _PKC * DW // 128
    pk = pl.pallas_call(
        body,
        grid=(pl.cdiv(V, _PKC),),
        in_specs=[pl.BlockSpec((D, _PKC), lambda i: (0, i))],
        out_specs=pl.BlockSpec((rows_per, 128), lambda i: (i, 0)),
        out_shape=jax.ShapeDtypeStruct((V * DW // 128, 128), jnp.int32),
    )(tbt)
    return pk.reshape(V, DW)


def kernel(obj, table, v, g, b):
    B, H = obj.shape
    obj_p = jnp.pad(obj.astype(jnp.int32), ((0, 0), (0, PAD - H)))
    vec = _sc_pool(obj_p.reshape(-1), _pack_table(table), B)
    vec = vec.reshape(B, D)
    return _tc_project(vec, v, g.reshape(A, 1), b.reshape(1, A))


# speedup vs baseline: 1.4506x; 1.0130x over previous
import functools

import jax
import jax.numpy as jnp
from jax import lax
from jax.experimental import pallas as pl
from jax.experimental.pallas import tpu as pltpu
from jax.experimental.pallas import tpu_sc as plsc

D = 32
DW = 16
A = 64
HIST = 50
PAD = 56
GROUP = 2
NC, NS = 2, 16
NW = NC * NS


def _sc_pool(obj_pad_flat, table_pk, B):
    E = B // NW
    NG = E // GROUP
    GI = GROUP * PAD
    NBUF = 8

    mesh = plsc.VectorSubcoreMesh(
        core_axis_name="c", subcore_axis_name="s",
        num_cores=NC, num_subcores=NS)

    @functools.partial(
        pl.kernel,
        out_type=jax.ShapeDtypeStruct((B * D,), jnp.float32),
        mesh=mesh,
        scratch_types=[
            pltpu.VMEM((E * PAD,), jnp.int32),
            pltpu.VMEM((NBUF, GI, DW), jnp.int32),
            pltpu.VMEM((E * D,), jnp.float32),
            [pltpu.SemaphoreType.DMA] * NBUF,
        ],
        compiler_params=pltpu.CompilerParams(use_tc_tiling_on_sc=False,
                                             needs_layout_passes=False),
    )
    def sc_kernel(obj_hbm, table_hbm, out_hbm, idx_v, rows_v, out_v, sems):
        wid = lax.axis_index("c") * NS + lax.axis_index("s")

        pltpu.sync_copy(obj_hbm.at[pl.ds(wid * (E * PAD), E * PAD)], idx_v)

        def gather(gg, buf):
            off = pl.multiple_of(gg * GI, 8)
            return pltpu.make_async_copy(
                table_hbm.at[idx_v.at[pl.ds(off, GI)]],
                rows_v.at[buf], sems[buf])

        himask = jnp.full((16,), -65536, dtype=jnp.int32)

        def unpack_row(w):
            lo = plsc.bitcast(jnp.left_shift(w, 16), jnp.float32)
            hi = plsc.bitcast(jnp.bitwise_and(w, himask), jnp.float32)
            return lo, hi

        def reduce_group(gg, buf):
            for e in range(GROUP):
                r0 = e * PAD
                acc_lo, acc_hi = unpack_row(rows_v[buf, r0, :])
                for j in range(1, HIST):
                    lo, hi = unpack_row(rows_v[buf, r0 + j, :])
                    acc_lo = acc_lo + lo
                    acc_hi = acc_hi + hi
                row = (gg * GROUP + e) * D
                out_v[pl.ds(pl.multiple_of(row, D), 16)] = acc_lo
                out_v[pl.ds(pl.multiple_of(row + 16, 16), 16)] = acc_hi

        for b in range(NBUF):
            gather(b, b).start()

        @pl.loop(0, NG, step=NBUF)
        def _outer(g):
            for buf in range(NBUF):
                gg = g + buf
                gather(gg, buf).wait()

                @pl.when(gg + NBUF < NG)
                def _start_next():
                    gather(gg + NBUF, buf).start()

                reduce_group(gg, buf)

        pltpu.sync_copy(out_v, out_hbm.at[pl.ds(wid * (E * D), E * D)])

    return sc_kernel(obj_pad_flat, table_pk)


def _tc_project(vec, v, g2, b2):
    B = vec.shape[0]

    def body(vec_ref, v_ref, g_ref, b_ref, o_ref):
        vv = v_ref[...]
        norm = jnp.sqrt(jnp.sum(vv * vv, axis=1, keepdims=True))
        w = (g_ref[...] / norm) * (1.0 / HIST) * vv
        o_ref[...] = lax.dot_general(
            w, vec_ref[...], (((1,), (1,)), ((), ())),
            preferred_element_type=jnp.float32) + b_ref[...]

    return pl.pallas_call(
        body,
        out_shape=jax.ShapeDtypeStruct((A, B), jnp.float32),
    )(vec, v, g2, b2).T


_PKC = 2048


def _pack_table(table):
    V = table.shape[0]
    tbt = table.T

    def body(x_ref, o_ref):
        x = x_ref[...]
        ul = lax.bitcast_convert_type(x[:DW, :], jnp.uint32)
        uh = lax.bitcast_convert_type(x[DW:, :], jnp.uint32)
        rl = (ul + jnp.uint32(0x7FFF) + ((ul >> 16) & jnp.uint32(1))) >> 16
        th = uh + jnp.uint32(0x7FFF) + ((uh >> 16) & jnp.uint32(1))
        w = (th & jnp.uint32(0xFFFF0000)) | rl
        wt = jnp.transpose(lax.bitcast_convert_type(w, jnp.int32))
        w3 = wt.reshape(_PKC // 8, 8, DW)
        for j in range(8):
            o_ref[:, pl.ds(j * DW, DW)] = w3[:, j, :]

    rows_per = _PKC * DW // 128
    pk = pl.pallas_call(
        body,
        grid=(pl.cdiv(V, _PKC),),
        in_specs=[pl.BlockSpec((D, _PKC), lambda i: (0, i))],
        out_specs=pl.BlockSpec((rows_per, 128), lambda i: (i, 0)),
        out_shape=jax.ShapeDtypeStruct((V * DW // 128, 128), jnp.int32),
    )(tbt)
    return pk.reshape(V, DW)


def kernel(obj, table, v, g, b):
    B, H = obj.shape
    obj_p = jnp.pad(obj.astype(jnp.int32), ((0, 0), (0, PAD - H)))
    vec = _sc_pool(obj_p.reshape(-1), _pack_table(table), B)
    vec = vec.reshape(B, D)
    return _tc_project(vec, v, g.reshape(A, 1), b.reshape(A, 1))
